# baseline (speedup 1.0000x reference)
import functools

import jax
import jax.numpy as jnp
from jax import lax
from jax.experimental import pallas as pl
from jax.experimental.pallas import tpu as pltpu
from jax.experimental.pallas import tpu_sc as plsc

_B, _D, _Q = 256, 128, 65536
_SCALE = 10.0
_BQ = 4096
_NMEM = _Q // _BQ
_NGRID = _NMEM + 1

_NC, _NS = 2, 16
_NW = _NC * _NS
_RW = _Q // _NW


def _tc_body(q_ref, ksf_ref, kdf1_ref, kdf2_ref, mem_ref, out_ref,
             prev_row, t_scr):
    j = pl.program_id(0)
    r = pl.program_id(1)

    @pl.when(r == 0)
    def _():
        m = mem_ref[...]
        m_shift = jnp.concatenate([prev_row[...], m[: _BQ - 1, :]], axis=0)
        prev_row[...] = m[_BQ - 1 : _BQ, :]
        t_scr[...] = jax.lax.dot_general(
            q_ref[...], m_shift,
            dimension_numbers=(((1,), (1,)), ((), ())),
            preferred_element_type=jnp.float32,
        ) * _SCALE

    out_ref[...] = t_scr[...]

    @pl.when(j == 0)
    def _():
        q = q_ref[...]
        p_sf = jnp.sum(q * ksf_ref[...], axis=1, keepdims=True)
        p_df1 = jnp.sum(q * kdf1_ref[...], axis=1, keepdims=True)
        p_df2 = jnp.sum(q * kdf2_ref[...], axis=1, keepdims=True)
        pos = jnp.where(r == 0, p_sf, jnp.where(r == 1, p_df1, p_df2))
        out_ref[:, 0:1] = pos * _SCALE


def _rep_spec():
    return pl.BlockSpec((_B, _D), lambda j, r: (0, 0))


def _tc_logits(q, k_sf, k_df1, k_df2, memory):
    return pl.pallas_call(
        _tc_body,
        grid=(_NGRID, 3),
        in_specs=[
            _rep_spec(), _rep_spec(), _rep_spec(), _rep_spec(),
            pl.BlockSpec((_BQ, _D),
                         lambda j, r: (jnp.minimum(j, _NMEM - 1), 0)),
        ],
        out_specs=pl.BlockSpec((_B, _BQ), lambda j, r: (r, j)),
        out_shape=jax.ShapeDtypeStruct((3 * _B, _Q + 1), jnp.float32),
        scratch_shapes=[
            pltpu.VMEM((1, _D), jnp.float32),
            pltpu.VMEM((_B, _BQ), jnp.float32),
        ],
        compiler_params=pltpu.CompilerParams(
            dimension_semantics=("arbitrary", "arbitrary"),
        ),
    )(q, k_sf, k_df1, k_df2, memory)


_CH = 256
_NCH = _RW // _CH


def _sc_body(mem_hbm, kall_hbm, out_hbm, buf, in_sems, out_sems, ow_sem):
    wid = lax.axis_index("s") * _NC + lax.axis_index("c")
    base = wid * _RW

    def cin(c):
        return pltpu.make_async_copy(
            mem_hbm.at[pl.ds(base + c * _CH, _CH)],
            buf.at[c % 2], in_sems.at[c % 2])

    def cout(c):
        return pltpu.make_async_copy(
            buf.at[c % 2],
            out_hbm.at[pl.ds(base + c * _CH, _CH)], out_sems.at[c % 2])

    cin(0).start()
    for c in range(_NCH):
        cin(c).wait()
        cout(c).start()
        if c + 1 < _NCH:
            if c >= 1:
                cout(c - 1).wait()
            cin(c + 1).start()
    cout(_NCH - 2).wait()
    cout(_NCH - 1).wait()

    @pl.when(wid == 0)
    def _():
        pltpu.async_copy(kall_hbm.at[...], buf.at[0], ow_sem).wait()
        pltpu.async_copy(buf.at[0], out_hbm.at[pl.ds(0, _B)], ow_sem).wait()


def _sc_queue_update(memory, k_all):
    mesh = plsc.VectorSubcoreMesh(core_axis_name="c", subcore_axis_name="s")
    return pl.kernel(
        _sc_body,
        mesh=mesh,
        out_type=jax.ShapeDtypeStruct((_Q, _D), jnp.float32),
        scratch_types=[
            pltpu.VMEM((2, _CH, _D), jnp.float32),
            pltpu.SemaphoreType.DMA((2,)),
            pltpu.SemaphoreType.DMA((2,)),
            pltpu.SemaphoreType.DMA,
        ],
    )(memory, k_all)


def kernel(q, k_sf, k_df1, k_df2, k_all, memory):
    out = _tc_logits(q, k_sf, k_df1, k_df2, memory)
    new_memory = _sc_queue_update(memory, k_all)
    return out, new_memory


# speedup vs baseline: 1.0949x; 1.0949x over previous
import functools

import jax
import jax.numpy as jnp
from jax import lax
from jax.experimental import pallas as pl
from jax.experimental.pallas import tpu as pltpu
from jax.experimental.pallas import tpu_sc as plsc

_B, _D, _Q = 256, 128, 65536
_SCALE = 10.0
_BQ = 4096
_NMEM = _Q // _BQ
_NGRID = _NMEM + 1

_NC, _NS = 2, 16
_NW = _NC * _NS
_RW = _Q // _NW


def _tc_body(q_ref, ksf_ref, kdf1_ref, kdf2_ref, mem_ref, out_ref, prev_row):
    j = pl.program_id(0)

    m = mem_ref[...]
    m_shift = jnp.concatenate([prev_row[...], m[: _BQ - 1, :]], axis=0)
    prev_row[...] = m[_BQ - 1 : _BQ, :]

    t = jax.lax.dot_general(
        q_ref[...], m_shift,
        dimension_numbers=(((1,), (1,)), ((), ())),
        preferred_element_type=jnp.float32,
    ) * _SCALE
    out_ref[0:_B, :] = t
    out_ref[_B:2 * _B, :] = t
    out_ref[2 * _B:3 * _B, :] = t

    @pl.when(j == 0)
    def _():
        q = q_ref[...]
        p_sf = jnp.sum(q * ksf_ref[...], axis=1, keepdims=True)
        p_df1 = jnp.sum(q * kdf1_ref[...], axis=1, keepdims=True)
        p_df2 = jnp.sum(q * kdf2_ref[...], axis=1, keepdims=True)
        pos = jnp.concatenate([p_sf, p_df1, p_df2], axis=0) * _SCALE
        out_ref[:, 0:1] = pos


def _rep_spec():
    return pl.BlockSpec((_B, _D), lambda j: (0, 0))


def _tc_logits(q, k_sf, k_df1, k_df2, memory):
    return pl.pallas_call(
        _tc_body,
        grid=(_NGRID,),
        in_specs=[
            _rep_spec(), _rep_spec(), _rep_spec(), _rep_spec(),
            pl.BlockSpec((_BQ, _D), lambda j: (jnp.minimum(j, _NMEM - 1), 0)),
        ],
        out_specs=pl.BlockSpec((3 * _B, _BQ), lambda j: (0, j)),
        out_shape=jax.ShapeDtypeStruct((3 * _B, _Q + 1), jnp.float32),
        scratch_shapes=[pltpu.VMEM((1, _D), jnp.float32)],
        compiler_params=pltpu.CompilerParams(
            dimension_semantics=("arbitrary",),
        ),
    )(q, k_sf, k_df1, k_df2, memory)


_CH = 256
_NCH = _RW // _CH


def _sc_body(mem_hbm, kall_hbm, out_hbm, buf, in_sems, out_sems, ow_sem):
    wid = lax.axis_index("s") * _NC + lax.axis_index("c")
    base = wid * _RW

    def cin(c):
        return pltpu.make_async_copy(
            mem_hbm.at[pl.ds(base + c * _CH, _CH)],
            buf.at[c % 2], in_sems.at[c % 2])

    def cout(c):
        return pltpu.make_async_copy(
            buf.at[c % 2],
            out_hbm.at[pl.ds(base + c * _CH, _CH)], out_sems.at[c % 2])

    cin(0).start()
    for c in range(_NCH):
        cin(c).wait()
        cout(c).start()
        if c + 1 < _NCH:
            if c >= 1:
                cout(c - 1).wait()
            cin(c + 1).start()
    cout(_NCH - 2).wait()
    cout(_NCH - 1).wait()

    @pl.when(wid == 0)
    def _():
        pltpu.async_copy(kall_hbm.at[...], buf.at[0], ow_sem).wait()
        pltpu.async_copy(buf.at[0], out_hbm.at[pl.ds(0, _B)], ow_sem).wait()


def _sc_queue_update(memory, k_all):
    mesh = plsc.VectorSubcoreMesh(core_axis_name="c", subcore_axis_name="s")
    return pl.kernel(
        _sc_body,
        mesh=mesh,
        out_type=jax.ShapeDtypeStruct((_Q, _D), jnp.float32),
        scratch_types=[
            pltpu.VMEM((2, _CH, _D), jnp.float32),
            pltpu.SemaphoreType.DMA((2,)),
            pltpu.SemaphoreType.DMA((2,)),
            pltpu.SemaphoreType.DMA,
        ],
    )(memory, k_all)


def kernel(q, k_sf, k_df1, k_df2, k_all, memory):
    out = _tc_logits(q, k_sf, k_df1, k_df2, memory)
    new_memory = _sc_queue_update(memory, k_all)
    return out, new_memory


# speedup vs baseline: 2.7806x; 2.5397x over previous
import functools

import jax
import jax.numpy as jnp
from jax import lax
from jax.experimental import pallas as pl
from jax.experimental.pallas import tpu as pltpu
from jax.experimental.pallas import tpu_sc as plsc

_B, _D, _Q = 256, 128, 65536
_SCALE = 10.0
_BQ = 4096
_NMEM = _Q // _BQ
_NGRID = _NMEM + 1

_NC, _NS = 2, 16
_NW = _NC * _NS
_RW = _Q // _NW


def _tc_body(q_ref, ksf_ref, kdf1_ref, kdf2_ref, mem_ref, out_ref, prev_row):
    j = pl.program_id(0)

    m = mem_ref[...]
    m_shift = jnp.concatenate([prev_row[...], m[: _BQ - 1, :]], axis=0)
    prev_row[...] = m[_BQ - 1 : _BQ, :]

    t = jax.lax.dot_general(
        m_shift, q_ref[...],
        dimension_numbers=(((1,), (1,)), ((), ())),
        preferred_element_type=jnp.float32,
    ) * _SCALE
    out_ref[:, 0:_B] = t
    out_ref[:, _B:2 * _B] = t
    out_ref[:, 2 * _B:3 * _B] = t

    @pl.when(j == 0)
    def _():
        q = q_ref[...]
        p_sf = jnp.sum(q * ksf_ref[...], axis=1, keepdims=True)
        p_df1 = jnp.sum(q * kdf1_ref[...], axis=1, keepdims=True)
        p_df2 = jnp.sum(q * kdf2_ref[...], axis=1, keepdims=True)
        pos = jnp.concatenate([p_sf, p_df1, p_df2], axis=0) * _SCALE
        out_ref[0:1, :] = pos.reshape(1, 3 * _B)


def _rep_spec():
    return pl.BlockSpec((_B, _D), lambda j: (0, 0))


def _tc_logits(q, k_sf, k_df1, k_df2, memory):
    return pl.pallas_call(
        _tc_body,
        grid=(_NGRID,),
        in_specs=[
            _rep_spec(), _rep_spec(), _rep_spec(), _rep_spec(),
            pl.BlockSpec((_BQ, _D), lambda j: (jnp.minimum(j, _NMEM - 1), 0)),
        ],
        out_specs=pl.BlockSpec((_BQ, 3 * _B), lambda j: (j, 0)),
        out_shape=jax.ShapeDtypeStruct((_Q + 1, 3 * _B), jnp.float32),
        scratch_shapes=[pltpu.VMEM((1, _D), jnp.float32)],
        compiler_params=pltpu.CompilerParams(
            dimension_semantics=("arbitrary",),
        ),
    )(q, k_sf, k_df1, k_df2, memory)


_CH = 256
_NCH = _RW // _CH


def _sc_body(mem_hbm, kall_hbm, out_hbm, buf, in_sems, out_sems, ow_sem):
    wid = lax.axis_index("s") * _NC + lax.axis_index("c")
    base = wid * _RW

    def cin(c):
        return pltpu.make_async_copy(
            mem_hbm.at[pl.ds(base + c * _CH, _CH)],
            buf.at[c % 2], in_sems.at[c % 2])

    def cout(c):
        return pltpu.make_async_copy(
            buf.at[c % 2],
            out_hbm.at[pl.ds(base + c * _CH, _CH)], out_sems.at[c % 2])

    cin(0).start()
    for c in range(_NCH):
        cin(c).wait()
        cout(c).start()
        if c + 1 < _NCH:
            if c >= 1:
                cout(c - 1).wait()
            cin(c + 1).start()
    cout(_NCH - 2).wait()
    cout(_NCH - 1).wait()

    @pl.when(wid == 0)
    def _():
        pltpu.async_copy(kall_hbm.at[...], buf.at[0], ow_sem).wait()
        pltpu.async_copy(buf.at[0], out_hbm.at[pl.ds(0, _B)], ow_sem).wait()


def _sc_queue_update(memory, k_all):
    mesh = plsc.VectorSubcoreMesh(core_axis_name="c", subcore_axis_name="s")
    return pl.kernel(
        _sc_body,
        mesh=mesh,
        out_type=jax.ShapeDtypeStruct((_Q, _D), jnp.float32),
        scratch_types=[
            pltpu.VMEM((2, _CH, _D), jnp.float32),
            pltpu.SemaphoreType.DMA((2,)),
            pltpu.SemaphoreType.DMA((2,)),
            pltpu.SemaphoreType.DMA,
        ],
    )(memory, k_all)


def kernel(q, k_sf, k_df1, k_df2, k_all, memory):
    out_t = _tc_logits(q, k_sf, k_df1, k_df2, memory)
    new_memory = _sc_queue_update(memory, k_all)
    return out_t.T, new_memory


# speedup vs baseline: 2.8123x; 1.0114x over previous
import functools

import jax
import jax.numpy as jnp
from jax import lax
from jax.experimental import pallas as pl
from jax.experimental.pallas import tpu as pltpu
from jax.experimental.pallas import tpu_sc as plsc

_B, _D, _Q = 256, 128, 65536
_SCALE = 10.0
_BQ = 8192
_NMEM = _Q // _BQ
_NGRID = _NMEM + 1

_NC, _NS = 2, 16
_NW = _NC * _NS
_RW = _Q // _NW


def _tc_body(q_ref, ksf_ref, kdf1_ref, kdf2_ref, mem_ref, out_ref, prev_row):
    j = pl.program_id(0)

    m = mem_ref[...]
    m_shift = jnp.concatenate([prev_row[...], m[: _BQ - 1, :]], axis=0)
    prev_row[...] = m[_BQ - 1 : _BQ, :]

    t = jax.lax.dot_general(
        m_shift, q_ref[...],
        dimension_numbers=(((1,), (1,)), ((), ())),
        preferred_element_type=jnp.float32,
    ) * _SCALE
    out_ref[:, 0:_B] = t
    out_ref[:, _B:2 * _B] = t
    out_ref[:, 2 * _B:3 * _B] = t

    @pl.when(j == 0)
    def _():
        q = q_ref[...]
        p_sf = jnp.sum(q * ksf_ref[...], axis=1, keepdims=True)
        p_df1 = jnp.sum(q * kdf1_ref[...], axis=1, keepdims=True)
        p_df2 = jnp.sum(q * kdf2_ref[...], axis=1, keepdims=True)
        pos = jnp.concatenate([p_sf, p_df1, p_df2], axis=0) * _SCALE
        out_ref[0:1, :] = pos.reshape(1, 3 * _B)


def _rep_spec():
    return pl.BlockSpec((_B, _D), lambda j: (0, 0))


def _tc_logits(q, k_sf, k_df1, k_df2, memory):
    return pl.pallas_call(
        _tc_body,
        grid=(_NGRID,),
        in_specs=[
            _rep_spec(), _rep_spec(), _rep_spec(), _rep_spec(),
            pl.BlockSpec((_BQ, _D), lambda j: (jnp.minimum(j, _NMEM - 1), 0)),
        ],
        out_specs=pl.BlockSpec((_BQ, 3 * _B), lambda j: (j, 0)),
        out_shape=jax.ShapeDtypeStruct((_Q + 1, 3 * _B), jnp.float32),
        scratch_shapes=[pltpu.VMEM((1, _D), jnp.float32)],
        compiler_params=pltpu.CompilerParams(
            dimension_semantics=("arbitrary",),
        ),
    )(q, k_sf, k_df1, k_df2, memory)


_CH = 256
_NCH = _RW // _CH


def _sc_body(mem_hbm, kall_hbm, out_hbm, buf, in_sems, out_sems, ow_sem):
    wid = lax.axis_index("s") * _NC + lax.axis_index("c")
    base = wid * _RW

    def cin(c):
        return pltpu.make_async_copy(
            mem_hbm.at[pl.ds(base + c * _CH, _CH)],
            buf.at[c % 2], in_sems.at[c % 2])

    def cout(c):
        return pltpu.make_async_copy(
            buf.at[c % 2],
            out_hbm.at[pl.ds(base + c * _CH, _CH)], out_sems.at[c % 2])

    cin(0).start()
    for c in range(_NCH):
        cin(c).wait()
        cout(c).start()
        if c + 1 < _NCH:
            if c >= 1:
                cout(c - 1).wait()
            cin(c + 1).start()
    cout(_NCH - 2).wait()
    cout(_NCH - 1).wait()

    @pl.when(wid == 0)
    def _():
        pltpu.async_copy(kall_hbm.at[...], buf.at[0], ow_sem).wait()
        pltpu.async_copy(buf.at[0], out_hbm.at[pl.ds(0, _B)], ow_sem).wait()


def _sc_queue_update(memory, k_all):
    mesh = plsc.VectorSubcoreMesh(core_axis_name="c", subcore_axis_name="s")
    return pl.kernel(
        _sc_body,
        mesh=mesh,
        out_type=jax.ShapeDtypeStruct((_Q, _D), jnp.float32),
        scratch_types=[
            pltpu.VMEM((2, _CH, _D), jnp.float32),
            pltpu.SemaphoreType.DMA((2,)),
            pltpu.SemaphoreType.DMA((2,)),
            pltpu.SemaphoreType.DMA,
        ],
    )(memory, k_all)


def kernel(q, k_sf, k_df1, k_df2, k_all, memory):
    out_t = _tc_logits(q, k_sf, k_df1, k_df2, memory)
    new_memory = _sc_queue_update(memory, k_all)
    return out_t.T, new_memory
